# Initial kernel scaffold; baseline (speedup 1.0000x reference)
import jax
import jax.numpy as jnp
from jax.experimental import pallas as pl


def kernel(xy):
    raise NotImplementedError("write your pallas kernel here")


# speedup vs baseline: 1.8283x; 1.8283x over previous
import jax
import jax.numpy as jnp
import numpy as np
from jax.experimental import pallas as pl

_NUM_BUCKETS = 512
_MAX_DIST = 3.0
_INV_RANGE = float(np.float32(1.0) / np.float32(2.0 * _MAX_DIST))


def _labels(v):
    return jnp.clip(
        ((v * _INV_RANGE + 0.5) * _NUM_BUCKETS).astype(jnp.int32),
        0, _NUM_BUCKETS - 1)


def _body(xy_ref, out_ref):
    seq = xy_ref.shape[-1]
    rows = jax.lax.broadcasted_iota(jnp.int32, (_NUM_BUCKETS, seq), 0)
    xl = _labels(xy_ref[0, 0:1, :])
    yl = _labels(xy_ref[0, 1:2, :])
    out_ref[0, :_NUM_BUCKETS] = (rows == xl).astype(jnp.float32)
    out_ref[0, _NUM_BUCKETS:] = (rows == yl).astype(jnp.float32)


def kernel(xy):
    bs, _, seq = xy.shape
    return pl.pallas_call(
        _body,
        grid=(bs,),
        in_specs=[pl.BlockSpec((1, 2, seq), lambda b: (b, 0, 0))],
        out_specs=pl.BlockSpec((1, 2 * _NUM_BUCKETS, seq), lambda b: (b, 0, 0)),
        out_shape=jax.ShapeDtypeStruct((bs, 2 * _NUM_BUCKETS, seq), jnp.float32),
    )(xy)
